# Initial kernel scaffold; baseline (speedup 1.0000x reference)
import jax
import jax.numpy as jnp
from jax.experimental import pallas as pl


def kernel(queries, keys):
    raise NotImplementedError("write your pallas kernel here")


# speedup vs baseline: 1.8522x; 1.8522x over previous
import jax
import jax.numpy as jnp
from jax.experimental import pallas as pl
from jax.experimental.pallas import tpu as pltpu

TOPK = 32
KB = 2048


def _block_topk_kernel(num_keys, q_ref, k_ref, vals_ref, idx_ref, qn_ref):
    i = pl.program_id(0)

    @pl.when(i == 0)
    def _():
        q = q_ref[...]
        qn_ref[...] = q / (jnp.sqrt(jnp.sum(q * q, axis=1, keepdims=True)) + 1e-8)

    kb = k_ref[...]
    kn = kb / (jnp.sqrt(jnp.sum(kb * kb, axis=1, keepdims=True)) + 1e-8)
    s = jnp.dot(qn_ref[...], kn.T, preferred_element_type=jnp.float32)
    iota = jax.lax.broadcasted_iota(jnp.int32, s.shape, 1)
    s = jnp.where(iota + i * KB < num_keys, s, -jnp.inf)
    vals = []
    idxs = []
    for _ in range(TOPK):
        m = jnp.max(s, axis=1, keepdims=True)
        p = jnp.min(jnp.where(s == m, iota, jnp.int32(KB)), axis=1, keepdims=True)
        vals.append(m)
        idxs.append(p)
        s = jnp.where(iota == p, -jnp.inf, s)
    vals_ref[0] = jnp.concatenate(vals, axis=1)
    idx_ref[0] = jnp.concatenate(idxs, axis=1) + i * KB


def _merge_kernel(v_ref, i_ref, ov_ref, oi_ref):
    v = v_ref[...]
    ix = i_ref[...]
    vals = []
    idxs = []
    for _ in range(TOPK):
        m = jnp.max(v, axis=1, keepdims=True)
        gi = jnp.min(jnp.where(v == m, ix, jnp.int32(2**31 - 1)), axis=1,
                     keepdims=True)
        vals.append(m)
        idxs.append(gi)
        v = jnp.where((v == m) & (ix == gi), -jnp.inf, v)
    ov_ref[...] = jnp.concatenate(vals, axis=1)
    oi_ref[...] = jnp.concatenate(idxs, axis=1)


def kernel(queries, keys):
    Q, D = queries.shape
    K = keys.shape[0]
    nkb = (K + KB - 1) // KB
    kpad = nkb * KB
    keys_p = jnp.pad(keys, ((0, kpad - K), (0, 0)))

    import functools
    bvals, bidx = pl.pallas_call(
        functools.partial(_block_topk_kernel, K),
        grid=(nkb,),
        in_specs=[
            pl.BlockSpec((Q, D), lambda i: (0, 0)),
            pl.BlockSpec((KB, D), lambda i: (i, 0)),
        ],
        out_specs=[
            pl.BlockSpec((1, Q, TOPK), lambda i: (i, 0, 0)),
            pl.BlockSpec((1, Q, TOPK), lambda i: (i, 0, 0)),
        ],
        out_shape=[
            jax.ShapeDtypeStruct((nkb, Q, TOPK), jnp.float32),
            jax.ShapeDtypeStruct((nkb, Q, TOPK), jnp.int32),
        ],
        scratch_shapes=[pltpu.VMEM((Q, D), jnp.float32)],
    )(queries, keys_p)

    ncand = nkb * TOPK
    cvals = bvals.transpose(1, 0, 2).reshape(Q, ncand)
    cidx = bidx.transpose(1, 0, 2).reshape(Q, ncand)

    vals, idx = pl.pallas_call(
        _merge_kernel,
        out_shape=[
            jax.ShapeDtypeStruct((Q, TOPK), jnp.float32),
            jax.ShapeDtypeStruct((Q, TOPK), jnp.int32),
        ],
    )(cvals, cidx)
    return vals, idx


# speedup vs baseline: 7.6990x; 4.1566x over previous
import functools

import jax
import jax.numpy as jnp
from jax import lax
from jax.experimental import pallas as pl
from jax.experimental.pallas import tpu as pltpu
from jax.experimental.pallas import tpu_sc as plsc

TOPK = 32
KB = 2048
CHUNK = 128
NCAND = 512
MGATH = 48
NWORK = 32


def _score_kernel(num_keys, q_ref, k_ref, s_ref, cm_ref):
    i = pl.program_id(0)
    s = jnp.dot(q_ref[...], k_ref[...].T,
                preferred_element_type=jnp.float32)
    iota = lax.broadcasted_iota(jnp.int32, s.shape, 1)
    s = jnp.where(iota + i * KB < num_keys, s, -1e30)
    s_ref[...] = s
    nq = s.shape[0]
    cm_ref[0] = jnp.max(s.reshape(nq, KB // CHUNK, CHUNK), axis=2)


def _thresh_kernel(cm3_ref, cm2_ref, t_ref):
    c3 = cm3_ref[...]
    nkb, nq, ch = c3.shape
    cm2 = jnp.transpose(c3, (1, 0, 2)).reshape(nq, nkb * ch)
    cm2_ref[...] = cm2
    c = cm2
    for _ in range(TOPK - 1):
        m = jnp.max(c, axis=1, keepdims=True)
        c = jnp.where(c == m, -jnp.inf, c)
    t_ref[...] = jnp.broadcast_to(jnp.max(c, axis=1, keepdims=True),
                                  t_ref.shape)


def _sc_filter_body(nchunks, s2_hbm, cm_hbm, t_hbm, cv_hbm, ci_hbm,
                    tv, cmv, clist, rows, cval, cidx, sem):
    core = lax.axis_index("c")
    sub = lax.axis_index("s")
    wid = sub * 2 + core
    qpw = cv_hbm.shape[0] // NWORK
    q0 = wid * qpw
    lane16 = lax.iota(jnp.int32, 16)

    def per_query(ql, _):
        q = q0 + ql
        pltpu.sync_copy(cm_hbm.at[q], cmv)
        pltpu.sync_copy(t_hbm.at[q], tv)
        t_v = tv[...]

        def scan_body(j, cnt):
            cm_v = cmv[pl.ds(j * 16, 16)]
            m = cm_v >= t_v
            ids = q * nchunks + j * 16 + lane16
            _, sids, om = plsc.sort_key_val(cm_v, ids, mask=m, descending=True)
            off = jnp.minimum(cnt, MGATH - 16)
            clist[pl.ds(off, 16)] = jnp.where(om, sids, q * nchunks)
            pc = plsc.all_reduce_population_count(m)
            return cnt + pc[0]

        def pre_body(jj, _c):
            clist[pl.ds(jj * 16, 16)] = jnp.full((16,), q * nchunks, jnp.int32)
            return _c
        lax.fori_loop(0, MGATH // 16, pre_body, 0)
        cnt = lax.fori_loop(0, nchunks // 16, scan_body, jnp.int32(0))

        pltpu.async_copy(s2_hbm.at[clist], rows, sem).wait()

        def init_body(jj, _c):
            cval[pl.ds(jj * 16, 16)] = jnp.full((16,), -1e30, jnp.float32)
            cidx[pl.ds(jj * 16, 16)] = jnp.zeros((16,), jnp.int32)
            return _c
        lax.fori_loop(0, (NCAND + 16) // 16, init_body, 0)

        def row_body(r, cur):
            rid_v = plsc.load_gather(clist, [jnp.full((16,), r, jnp.int32)])
            base_v = (rid_v - q * nchunks) * CHUNK
            valid = jnp.full((16,), r, jnp.int32) < jnp.full((16,), cnt,
                                                            jnp.int32)
            teff = jnp.where(valid, t_v, jnp.float32(1e30))

            def v_body(v, cur2):
                s_v = rows[r, pl.ds(v * 16, 16)]
                m = s_v >= teff
                sv, si, om = plsc.sort_key_val(
                    s_v, base_v + v * 16 + lane16, mask=m, descending=True)
                off = jnp.minimum(cur2, NCAND)
                cval[pl.ds(off, 16)] = jnp.where(om, sv, jnp.float32(-1e30))
                cidx[pl.ds(off, 16)] = jnp.where(om, si, jnp.int32(0))
                pc = plsc.all_reduce_population_count(m)
                return jnp.minimum(cur2 + pc[0], NCAND)

            return lax.fori_loop(0, CHUNK // 16, v_body, cur)

        lax.fori_loop(0, MGATH, row_body, jnp.int32(0))

        pltpu.sync_copy(cval.at[pl.ds(0, NCAND)], cv_hbm.at[q])
        pltpu.sync_copy(cidx.at[pl.ds(0, NCAND)], ci_hbm.at[q])
        return 0

    lax.fori_loop(0, qpw, per_query, 0)


def _merge_kernel(v_ref, i_ref, ov_ref, oi_ref):
    v = v_ref[...]
    ix = i_ref[...]
    vals = []
    idxs = []
    for _ in range(TOPK):
        m = jnp.max(v, axis=1, keepdims=True)
        gi = jnp.min(jnp.where(v == m, ix, jnp.int32(2**31 - 1)), axis=1,
                     keepdims=True)
        vals.append(m)
        idxs.append(gi)
        v = jnp.where((v == m) & (ix == gi), -jnp.inf, v)
    ov_ref[...] = jnp.concatenate(vals, axis=1)
    oi_ref[...] = jnp.concatenate(idxs, axis=1)


def kernel(queries, keys):
    Q, D = queries.shape
    K = keys.shape[0]
    nkb = (K + KB - 1) // KB
    kpad = nkb * KB
    nchunks = kpad // CHUNK
    qn = queries / (jnp.linalg.norm(queries, axis=-1, keepdims=True) + 1e-8)
    kn = keys / (jnp.linalg.norm(keys, axis=-1, keepdims=True) + 1e-8)
    keys_p = jnp.pad(kn, ((0, kpad - K), (0, 0)))

    scores, cm3 = pl.pallas_call(
        functools.partial(_score_kernel, K),
        grid=(nkb,),
        in_specs=[
            pl.BlockSpec((Q, D), lambda i: (0, 0)),
            pl.BlockSpec((KB, D), lambda i: (i, 0)),
        ],
        out_specs=[
            pl.BlockSpec((Q, KB), lambda i: (0, i)),
            pl.BlockSpec((1, Q, KB // CHUNK), lambda i: (i, 0, 0)),
        ],
        out_shape=[
            jax.ShapeDtypeStruct((Q, kpad), jnp.float32),
            jax.ShapeDtypeStruct((nkb, Q, KB // CHUNK), jnp.float32),
        ],
    )(qn, keys_p)

    cm2, trep = pl.pallas_call(
        _thresh_kernel,
        out_shape=[
            jax.ShapeDtypeStruct((Q, nchunks), jnp.float32),
            jax.ShapeDtypeStruct((Q, 16), jnp.float32),
        ],
    )(cm3)

    s2 = scores.reshape(Q * nchunks, CHUNK)

    sc_fn = functools.partial(
        pl.kernel,
        out_type=(jax.ShapeDtypeStruct((Q, NCAND), jnp.float32),
                  jax.ShapeDtypeStruct((Q, NCAND), jnp.int32)),
        mesh=plsc.VectorSubcoreMesh(core_axis_name="c", subcore_axis_name="s"),
        compiler_params=pltpu.CompilerParams(needs_layout_passes=False),
        scratch_types=[
            pltpu.VMEM((16,), jnp.float32),
            pltpu.VMEM((nchunks,), jnp.float32),
            pltpu.VMEM((MGATH,), jnp.int32),
            pltpu.VMEM((MGATH, CHUNK), jnp.float32),
            pltpu.VMEM((NCAND + 16,), jnp.float32),
            pltpu.VMEM((NCAND + 16,), jnp.int32),
            pltpu.SemaphoreType.DMA,
        ],
    )(functools.partial(_sc_filter_body, nchunks))
    cand_val, cand_idx = sc_fn(s2, cm2, trep)

    vals, idx = pl.pallas_call(
        _merge_kernel,
        out_shape=[
            jax.ShapeDtypeStruct((Q, TOPK), jnp.float32),
            jax.ShapeDtypeStruct((Q, TOPK), jnp.int32),
        ],
    )(cand_val, cand_idx)
    return vals, idx
